# baseline (speedup 1.0000x reference)
import functools

import jax
import jax.numpy as jnp
from jax import lax
from jax.experimental import pallas as pl
from jax.experimental.pallas import tpu as pltpu
from jax.experimental.pallas import tpu_sc as plsc

VOCAB = 1000000
EMBED = 64
NTOKENS = 200
BATCH = 4096

TOTAL = BATCH * NTOKENS
NUM_WORKERS = 32
PER_WORKER = TOTAL // NUM_WORKERS
CHUNK = NTOKENS
NCHUNKS = PER_WORKER // CHUNK
HALF0 = 96
HALF1 = 112
IDXS = 224
ROWS = 208
NHALF = 2 * NCHUNKS
NITER = NHALF + 2

_mesh = plsc.VectorSubcoreMesh(core_axis_name="c", subcore_axis_name="s")


@functools.partial(
    pl.kernel,
    mesh=_mesh,
    out_type=jax.ShapeDtypeStruct((TOTAL, EMBED), jnp.float32),
    scratch_types=[
        pltpu.VMEM((2 * IDXS,), jnp.int32),
        pltpu.VMEM((3 * ROWS, EMBED), jnp.float32),
        pltpu.VMEM((ROWS, EMBED), jnp.float32),
        pltpu.SemaphoreType.DMA,
        pltpu.SemaphoreType.DMA,
        pltpu.SemaphoreType.DMA,
    ],
    compiler_params=pltpu.CompilerParams(use_tc_tiling_on_sc=True),
)
def _embed_sc(tokens_hbm, table_hbm, pos_hbm, out_hbm,
              idx_v, rows_v, pos_v, sem_i, sem_r, sem_o):
    wid = lax.axis_index("s") * 2 + lax.axis_index("c")
    base = wid * PER_WORKER
    last = NCHUNKS - 1
    zeros16 = jnp.zeros((16,), jnp.int32)

    pltpu.sync_copy(pos_hbm, pos_v.at[pl.ds(0, CHUNK)])

    def idx_fetch(c, slot):
        c = jnp.minimum(c, last)
        pltpu.async_copy(tokens_hbm.at[pl.ds(base + c * CHUNK, CHUNK)],
                         idx_v.at[pl.ds(slot * IDXS, CHUNK)], sem_i)

    def idx_wait():
        pltpu.make_async_copy(tokens_hbm.at[pl.ds(0, CHUNK)],
                              idx_v.at[pl.ds(0, CHUNK)], sem_i).wait()

    def fire_half(islot, rslot, tok0, ntok):
        for k in range(tok0, tok0 + ntok, 16):
            tv = idx_v[pl.ds(islot + k, 16)]
            ts = [tv[i] for i in range(16)]
            for i in range(16):
                pltpu.async_copy(table_hbm.at[pl.ds(ts[i], 1)],
                                 rows_v.at[pl.ds(rslot + k + i, 1)], sem_r)

    def add_half(rslot, tok0, ntok):
        for r in range(tok0, tok0 + ntok):
            for c in range(EMBED // 16):
                sl = pl.ds(c * 16, 16)
                rows_v[rslot + r, sl] = rows_v[rslot + r, sl] + pos_v[r, sl]

    def drain_half(rslot, tok0, ntok):
        pltpu.make_async_copy(
            table_hbm.at[pl.ds(0, 16)],
            rows_v.at[pl.ds(rslot + tok0, ntok)], sem_r).wait()

    idx_fetch(0, 0)

    def body(g, carry):
        cf = lax.shift_right_logical(g, 1)
        hf = lax.bitwise_and(g, 1)
        ga = g - 2
        ca = lax.shift_right_logical(ga, 1)
        ha = lax.bitwise_and(ga, 1)
        fslot_i = lax.rem(cf, 2) * IDXS
        fslot_r = lax.rem(cf, 3) * ROWS
        aslot_r = lax.rem(ca, 3) * ROWS

        @pl.when(jnp.logical_and(g < NHALF, hf == 0))
        def _():
            idx_wait()
            idx_v[pl.ds(fslot_i + CHUNK, 16)] = zeros16

            @pl.when(cf >= 3)
            def _():
                pltpu.make_async_copy(
                    rows_v.at[pl.ds(0, CHUNK)],
                    out_hbm.at[pl.ds(0, CHUNK)], sem_o).wait()

            idx_fetch(cf + 1, lax.rem(cf + 1, 2))
            fire_half(fslot_i, fslot_r, 0, HALF0)

            @pl.when(ga >= 0)
            def _():
                drain_half(aslot_r, 0, HALF0)
                add_half(aslot_r, 0, HALF0)

        @pl.when(jnp.logical_and(g < NHALF, hf == 1))
        def _():
            fire_half(fslot_i, fslot_r, HALF0, HALF1)

            @pl.when(ga >= 0)
            def _():
                drain_half(aslot_r, HALF0, HALF1)
                add_half(aslot_r, HALF0, HALF1)
                pltpu.async_copy(
                    rows_v.at[pl.ds(aslot_r, CHUNK)],
                    out_hbm.at[pl.ds(base + ca * CHUNK, CHUNK)], sem_o)

        @pl.when(g >= NHALF)
        def _():
            @pl.when(ha == 0)
            def _():
                drain_half(aslot_r, 0, HALF0)
                add_half(aslot_r, 0, HALF0)

            @pl.when(ha == 1)
            def _():
                drain_half(aslot_r, HALF0, HALF1)
                add_half(aslot_r, HALF0, HALF1)
                pltpu.async_copy(
                    rows_v.at[pl.ds(aslot_r, CHUNK)],
                    out_hbm.at[pl.ds(base + ca * CHUNK, CHUNK)], sem_o)

        return carry

    lax.fori_loop(0, NITER, body, 0)

    idx_wait()
    for _ in range(3):
        pltpu.make_async_copy(rows_v.at[pl.ds(0, CHUNK)],
                              out_hbm.at[pl.ds(0, CHUNK)], sem_o).wait()


def kernel(tokens, input_embedding, position_embedding):
    flat = tokens.reshape(-1).astype(jnp.int32)
    out = _embed_sc(flat, input_embedding, position_embedding)
    return out.reshape(BATCH, NTOKENS, EMBED)


# speedup vs baseline: 1.0020x; 1.0020x over previous
import functools

import jax
import jax.numpy as jnp
from jax import lax
from jax.experimental import pallas as pl
from jax.experimental.pallas import tpu as pltpu
from jax.experimental.pallas import tpu_sc as plsc

VOCAB = 1000000
EMBED = 64
NTOKENS = 200
BATCH = 4096

TOTAL = BATCH * NTOKENS
NUM_WORKERS = 32
PER_WORKER = TOTAL // NUM_WORKERS
CHUNK = NTOKENS
NCHUNKS = PER_WORKER // CHUNK
HALF0 = 96
HALF1 = 112
IDXS = 224
ROWS = 208
NHALF = 2 * NCHUNKS
LAG = 4
NITER = NHALF + LAG

_mesh = plsc.VectorSubcoreMesh(core_axis_name="c", subcore_axis_name="s")


@functools.partial(
    pl.kernel,
    mesh=_mesh,
    out_type=jax.ShapeDtypeStruct((TOTAL, EMBED), jnp.float32),
    scratch_types=[
        pltpu.VMEM((2 * IDXS,), jnp.int32),
        pltpu.VMEM((3 * ROWS, EMBED), jnp.float32),
        pltpu.VMEM((ROWS, EMBED), jnp.float32),
        pltpu.SemaphoreType.DMA,
        pltpu.SemaphoreType.DMA,
        pltpu.SemaphoreType.DMA,
    ],
    compiler_params=pltpu.CompilerParams(use_tc_tiling_on_sc=True),
)
def _embed_sc(tokens_hbm, table_hbm, pos_hbm, out_hbm,
              idx_v, rows_v, pos_v, sem_i, sem_r, sem_o):
    wid = lax.axis_index("s") * 2 + lax.axis_index("c")
    base = wid * PER_WORKER
    last = NCHUNKS - 1
    zeros16 = jnp.zeros((16,), jnp.int32)

    pltpu.sync_copy(pos_hbm, pos_v.at[pl.ds(0, CHUNK)])

    def idx_fetch(c, slot):
        c = jnp.minimum(c, last)
        pltpu.async_copy(tokens_hbm.at[pl.ds(base + c * CHUNK, CHUNK)],
                         idx_v.at[pl.ds(slot * IDXS, CHUNK)], sem_i)

    def idx_wait():
        pltpu.make_async_copy(tokens_hbm.at[pl.ds(0, CHUNK)],
                              idx_v.at[pl.ds(0, CHUNK)], sem_i).wait()

    def fire_half(islot, rslot, tok0, ntok):
        for k in range(tok0, tok0 + ntok, 16):
            tv = idx_v[pl.ds(islot + k, 16)]
            ts = [tv[i] for i in range(16)]
            for i in range(16):
                pltpu.async_copy(table_hbm.at[pl.ds(ts[i], 1)],
                                 rows_v.at[pl.ds(rslot + k + i, 1)], sem_r)

    def add_half(rslot, tok0, ntok):
        for r in range(tok0, tok0 + ntok):
            for c in range(EMBED // 16):
                sl = pl.ds(c * 16, 16)
                rows_v[rslot + r, sl] = rows_v[rslot + r, sl] + pos_v[r, sl]

    def drain_half(rslot, tok0, ntok):
        for j in range(tok0, tok0 + ntok):
            pltpu.make_async_copy(
                table_hbm.at[pl.ds(0, 1)],
                rows_v.at[pl.ds(rslot + j, 1)], sem_r).wait()

    idx_fetch(0, 0)

    def body(g, carry):
        cf = lax.shift_right_logical(g, 1)
        hf = lax.bitwise_and(g, 1)
        ga = g - LAG
        ca = lax.shift_right_logical(ga, 1)
        ha = lax.bitwise_and(ga, 1)
        fslot_i = lax.rem(cf, 2) * IDXS
        fslot_r = lax.rem(cf, 3) * ROWS
        aslot_r = lax.rem(ca, 3) * ROWS

        @pl.when(jnp.logical_and(g < NHALF, hf == 0))
        def _():
            idx_wait()
            idx_v[pl.ds(fslot_i + CHUNK, 16)] = zeros16

            @pl.when(cf >= 3)
            def _():
                pltpu.make_async_copy(
                    rows_v.at[pl.ds(0, CHUNK)],
                    out_hbm.at[pl.ds(0, CHUNK)], sem_o).wait()

            idx_fetch(cf + 1, lax.rem(cf + 1, 2))
            fire_half(fslot_i, fslot_r, 0, HALF0)

            @pl.when(ga >= 0)
            def _():
                drain_half(aslot_r, 0, HALF0)
                add_half(aslot_r, 0, HALF0)

        @pl.when(jnp.logical_and(g < NHALF, hf == 1))
        def _():
            fire_half(fslot_i, fslot_r, HALF0, HALF1)

            @pl.when(ga >= 0)
            def _():
                drain_half(aslot_r, HALF0, HALF1)
                add_half(aslot_r, HALF0, HALF1)
                pltpu.async_copy(
                    rows_v.at[pl.ds(aslot_r, CHUNK)],
                    out_hbm.at[pl.ds(base + ca * CHUNK, CHUNK)], sem_o)

        @pl.when(g >= NHALF)
        def _():
            @pl.when(ha == 0)
            def _():
                drain_half(aslot_r, 0, HALF0)
                add_half(aslot_r, 0, HALF0)

            @pl.when(ha == 1)
            def _():
                drain_half(aslot_r, HALF0, HALF1)
                add_half(aslot_r, HALF0, HALF1)
                pltpu.async_copy(
                    rows_v.at[pl.ds(aslot_r, CHUNK)],
                    out_hbm.at[pl.ds(base + ca * CHUNK, CHUNK)], sem_o)

        return carry

    lax.fori_loop(0, NITER, body, 0)

    idx_wait()
    for _ in range(3):
        pltpu.make_async_copy(rows_v.at[pl.ds(0, CHUNK)],
                              out_hbm.at[pl.ds(0, CHUNK)], sem_o).wait()


def kernel(tokens, input_embedding, position_embedding):
    flat = tokens.reshape(-1).astype(jnp.int32)
    out = _embed_sc(flat, input_embedding, position_embedding)
    return out.reshape(BATCH, NTOKENS, EMBED)


# speedup vs baseline: 2.3873x; 2.3826x over previous
import functools

import jax
import jax.numpy as jnp
from jax import lax
from jax.experimental import pallas as pl
from jax.experimental.pallas import tpu as pltpu
from jax.experimental.pallas import tpu_sc as plsc

VOCAB = 1000000
EMBED = 64
NTOKENS = 200
BATCH = 4096

TOTAL = BATCH * NTOKENS
NUM_WORKERS = 32
PER_WORKER = TOTAL // NUM_WORKERS
CHUNK = NTOKENS
NCHUNKS = PER_WORKER // CHUNK
GROUPS = [16] * 12 + [8]
ADD_SPLITS = [15] * 12 + [20]

_mesh = plsc.VectorSubcoreMesh(core_axis_name="c", subcore_axis_name="s")


@functools.partial(
    pl.kernel,
    mesh=_mesh,
    out_type=jax.ShapeDtypeStruct((TOTAL, EMBED), jnp.float32),
    scratch_types=[
        pltpu.VMEM((CHUNK + 8,), jnp.int32),
        pltpu.VMEM((CHUNK + 8,), jnp.int32),
        pltpu.VMEM((CHUNK, EMBED), jnp.float32),
        pltpu.VMEM((CHUNK, EMBED), jnp.float32),
        pltpu.VMEM((CHUNK, EMBED), jnp.float32),
        pltpu.SemaphoreType.DMA,
        pltpu.SemaphoreType.DMA,
        pltpu.SemaphoreType.DMA,
        pltpu.SemaphoreType.DMA,
        pltpu.SemaphoreType.DMA,
        pltpu.SemaphoreType.DMA,
    ],
    compiler_params=pltpu.CompilerParams(use_tc_tiling_on_sc=True),
)
def _embed_sc(tokens_hbm, table_hbm, pos_hbm, out_hbm,
              idx_a, idx_b, rows_a, rows_b, pos_v,
              sem_ia, sem_ib, sem_ra, sem_rb, sem_oa, sem_ob):
    wid = lax.axis_index("s") * 2 + lax.axis_index("c")
    base = wid * PER_WORKER
    last = NCHUNKS - 1

    pltpu.sync_copy(pos_hbm, pos_v)

    def fire_group(idx_v, rows_v, sem, j, gsz):
        v = idx_v[pl.ds(j, 16)]
        ts = [v[i] for i in range(gsz)]
        for i in range(gsz):
            pltpu.async_copy(
                table_hbm.at[pl.ds(ts[i], 1)], rows_v.at[pl.ds(j + i, 1)], sem)

    def add_slice(rows_v, r0, nr):
        def row_body(r, c2):
            for cc in range(EMBED // 16):
                sl = pl.ds(cc * 16, 16)
                rows_v[r, sl] = rows_v[r, sl] + pos_v[r, sl]
            return c2

        lax.fori_loop(r0, r0 + nr, row_body, 0)

    def fire_interleaved(idx_f, rows_f, sem_f, rows_add):
        j = 0
        r0 = 0
        for gsz, nr in zip(GROUPS, ADD_SPLITS):
            fire_group(idx_f, rows_f, sem_f, j, gsz)
            add_slice(rows_add, r0, nr)
            j += gsz
            r0 += nr

    def fire_all(idx_v, rows_v, sem):
        j = 0
        for gsz in GROUPS:
            fire_group(idx_v, rows_v, sem, j, gsz)
            j += gsz

    def drain_rows(rows_v, sem):
        for j in range(CHUNK):
            pltpu.make_async_copy(
                table_hbm.at[pl.ds(0, 1)], rows_v.at[pl.ds(j, 1)], sem
            ).wait()

    def idx_fetch(c, idx_v, sem):
        c = jnp.minimum(c, last)
        pltpu.async_copy(tokens_hbm.at[pl.ds(base + c * CHUNK, CHUNK)],
                         idx_v.at[pl.ds(0, CHUNK)], sem)

    def idx_wait(idx_v, sem):
        pltpu.make_async_copy(tokens_hbm.at[pl.ds(0, CHUNK)],
                              idx_v.at[pl.ds(0, CHUNK)], sem).wait()

    def out_start(rows_v, c, sem):
        pltpu.async_copy(rows_v, out_hbm.at[pl.ds(base + c * CHUNK, CHUNK)], sem)

    def out_wait(rows_v, sem):
        pltpu.make_async_copy(rows_v, out_hbm.at[pl.ds(0, CHUNK)], sem).wait()

    pltpu.sync_copy(tokens_hbm.at[pl.ds(base, CHUNK)], idx_a.at[pl.ds(0, CHUNK)])
    fire_all(idx_a, rows_a, sem_ra)
    idx_fetch(1, idx_b, sem_ib)

    def pair_body(g, carry):
        ca = 2 * g

        idx_wait(idx_b, sem_ib)

        @pl.when(g > 0)
        def _():
            out_wait(rows_b, sem_ob)

        idx_fetch(ca + 2, idx_a, sem_ia)
        drain_rows(rows_a, sem_ra)
        fire_interleaved(idx_b, rows_b, sem_rb, rows_a)
        out_start(rows_a, ca, sem_oa)

        idx_wait(idx_a, sem_ia)
        out_wait(rows_a, sem_oa)
        idx_fetch(ca + 3, idx_b, sem_ib)
        drain_rows(rows_b, sem_rb)
        fire_interleaved(idx_a, rows_a, sem_ra, rows_b)
        out_start(rows_b, ca + 1, sem_ob)
        return carry

    lax.fori_loop(0, NCHUNKS // 2, pair_body, 0)

    idx_wait(idx_b, sem_ib)
    drain_rows(rows_a, sem_ra)
    out_wait(rows_b, sem_ob)


def kernel(tokens, input_embedding, position_embedding):
    flat = tokens.reshape(-1).astype(jnp.int32)
    out = _embed_sc(flat, input_embedding, position_embedding)
    return out.reshape(BATCH, NTOKENS, EMBED)


# speedup vs baseline: 2.7156x; 1.1375x over previous
import functools

import jax
import jax.numpy as jnp
from jax import lax
from jax.experimental import pallas as pl
from jax.experimental.pallas import tpu as pltpu
from jax.experimental.pallas import tpu_sc as plsc

VOCAB = 1000000
EMBED = 64
NTOKENS = 200
BATCH = 4096

TOTAL = BATCH * NTOKENS
NUM_WORKERS = 32
PER_WORKER = TOTAL // NUM_WORKERS
CHUNK = NTOKENS
NCHUNKS = PER_WORKER // CHUNK
GROUPS = [16] * 12 + [8]
ADD_SPLITS = [15] * 12 + [20]

_mesh = plsc.VectorSubcoreMesh(core_axis_name="c", subcore_axis_name="s")


@functools.partial(
    pl.kernel,
    mesh=_mesh,
    out_type=jax.ShapeDtypeStruct((TOTAL, EMBED), jnp.float32),
    scratch_types=[
        pltpu.VMEM((CHUNK + 8,), jnp.int32),
        pltpu.VMEM((CHUNK + 8,), jnp.int32),
        pltpu.VMEM((CHUNK, EMBED), jnp.float32),
        pltpu.VMEM((CHUNK, EMBED), jnp.float32),
        pltpu.VMEM((CHUNK, EMBED), jnp.float32),
        pltpu.SemaphoreType.DMA,
        pltpu.SemaphoreType.DMA,
        pltpu.SemaphoreType.DMA,
        pltpu.SemaphoreType.DMA,
        pltpu.SemaphoreType.DMA,
        pltpu.SemaphoreType.DMA,
    ],
    compiler_params=pltpu.CompilerParams(use_tc_tiling_on_sc=True),
)
def _embed_sc(tokens_hbm, table_hbm, pos_hbm, out_hbm,
              idx_a, idx_b, rows_a, rows_b, pos_v,
              sem_ia, sem_ib, sem_ra, sem_rb, sem_oa, sem_ob):
    wid = lax.axis_index("s") * 2 + lax.axis_index("c")
    base = wid * PER_WORKER
    last = NCHUNKS - 1

    pltpu.sync_copy(pos_hbm, pos_v)

    def fire_group(idx_v, rows_v, sem, j, gsz):
        v = idx_v[pl.ds(j, 16)]
        ts = [v[i] for i in range(gsz)]
        for i in range(gsz):
            t = ts[i]
            pltpu.async_copy(
                table_hbm.at[lax.shift_right_logical(t, 3),
                             pl.ds(lax.bitwise_and(t, 7), 1)],
                rows_v.at[pl.ds(j + i, 1)], sem)

    def add_slice(rows_v, r0, nr):
        def row_body(r, c2):
            for cc in range(EMBED // 16):
                sl = pl.ds(cc * 16, 16)
                rows_v[r, sl] = rows_v[r, sl] + pos_v[r, sl]
            return c2

        lax.fori_loop(r0, r0 + nr, row_body, 0)

    def fire_interleaved(idx_f, rows_f, sem_f, rows_add):
        j = 0
        r0 = 0
        for gsz, nr in zip(GROUPS, ADD_SPLITS):
            fire_group(idx_f, rows_f, sem_f, j, gsz)
            add_slice(rows_add, r0, nr)
            j += gsz
            r0 += nr

    def fire_all(idx_v, rows_v, sem):
        j = 0
        for gsz in GROUPS:
            fire_group(idx_v, rows_v, sem, j, gsz)
            j += gsz

    def drain_rows(rows_v, sem):
        for j in range(CHUNK):
            pltpu.make_async_copy(
                table_hbm.at[0, pl.ds(0, 1)], rows_v.at[pl.ds(j, 1)], sem
            ).wait()

    def idx_fetch(c, idx_v, sem):
        c = jnp.minimum(c, last)
        pltpu.async_copy(tokens_hbm.at[pl.ds(base + c * CHUNK, CHUNK)],
                         idx_v.at[pl.ds(0, CHUNK)], sem)

    def idx_wait(idx_v, sem):
        pltpu.make_async_copy(tokens_hbm.at[pl.ds(0, CHUNK)],
                              idx_v.at[pl.ds(0, CHUNK)], sem).wait()

    def out_start(rows_v, c, sem):
        pltpu.async_copy(rows_v, out_hbm.at[pl.ds(base + c * CHUNK, CHUNK)], sem)

    def out_wait(rows_v, sem):
        pltpu.make_async_copy(rows_v, out_hbm.at[pl.ds(0, CHUNK)], sem).wait()

    pltpu.sync_copy(tokens_hbm.at[pl.ds(base, CHUNK)], idx_a.at[pl.ds(0, CHUNK)])
    fire_all(idx_a, rows_a, sem_ra)
    idx_fetch(1, idx_b, sem_ib)

    def pair_body(g, carry):
        ca = 2 * g

        idx_wait(idx_b, sem_ib)

        @pl.when(g > 0)
        def _():
            out_wait(rows_b, sem_ob)

        idx_fetch(ca + 2, idx_a, sem_ia)
        drain_rows(rows_a, sem_ra)
        fire_interleaved(idx_b, rows_b, sem_rb, rows_a)
        out_start(rows_a, ca, sem_oa)

        idx_wait(idx_a, sem_ia)
        out_wait(rows_a, sem_oa)
        idx_fetch(ca + 3, idx_b, sem_ib)
        drain_rows(rows_b, sem_rb)
        fire_interleaved(idx_a, rows_a, sem_ra, rows_b)
        out_start(rows_b, ca + 1, sem_ob)
        return carry

    lax.fori_loop(0, NCHUNKS // 2, pair_body, 0)

    idx_wait(idx_b, sem_ib)
    drain_rows(rows_a, sem_ra)
    out_wait(rows_b, sem_ob)


def kernel(tokens, input_embedding, position_embedding):
    flat = tokens.reshape(-1).astype(jnp.int32)
    table3 = input_embedding.reshape(VOCAB // 8, 8, EMBED)
    out = _embed_sc(flat, table3, position_embedding)
    return out.reshape(BATCH, NTOKENS, EMBED)


# speedup vs baseline: 3.2045x; 1.1800x over previous
import functools

import jax
import jax.numpy as jnp
from jax import lax
from jax.experimental import pallas as pl
from jax.experimental.pallas import tpu as pltpu
from jax.experimental.pallas import tpu_sc as plsc

VOCAB = 1000000
EMBED = 64
NTOKENS = 200
BATCH = 4096

TOTAL = BATCH * NTOKENS
NUM_WORKERS = 32
PER_WORKER = TOTAL // NUM_WORKERS
CHUNK = NTOKENS
NCHUNKS = PER_WORKER // CHUNK
GROUPS = [16] * 12 + [8]
ADD_SPLITS = [15] * 12 + [20]

_mesh = plsc.VectorSubcoreMesh(core_axis_name="c", subcore_axis_name="s")


@functools.partial(
    pl.kernel,
    mesh=_mesh,
    out_type=jax.ShapeDtypeStruct((TOTAL, EMBED), jnp.float32),
    scratch_types=[
        pltpu.VMEM((CHUNK + 8,), jnp.int32),
        pltpu.VMEM((CHUNK + 8,), jnp.int32),
        pltpu.VMEM((CHUNK, EMBED), jnp.float32),
        pltpu.VMEM((CHUNK, EMBED), jnp.float32),
        pltpu.VMEM((CHUNK, EMBED), jnp.float32),
        pltpu.SemaphoreType.DMA,
        pltpu.SemaphoreType.DMA,
        pltpu.SemaphoreType.DMA,
        pltpu.SemaphoreType.DMA,
        pltpu.SemaphoreType.DMA,
        pltpu.SemaphoreType.DMA,
    ],
    compiler_params=pltpu.CompilerParams(use_tc_tiling_on_sc=True),
)
def _embed_sc(tokens_hbm, table_hbm, pos_hbm, out_hbm,
              idx_a, idx_b, rows_a, rows_b, pos_v,
              sem_ia, sem_ib, sem_ra, sem_rb, sem_oa, sem_ob):
    wid = lax.axis_index("s") * 2 + lax.axis_index("c")
    base = wid * PER_WORKER
    last = NCHUNKS - 1

    pltpu.sync_copy(pos_hbm, pos_v)

    def fire_group(idx_v, rows_v, sem, j, gsz):
        v = idx_v[pl.ds(j, 16)]
        ts = [v[i] for i in range(gsz)]
        for i in range(gsz):
            t = ts[i]
            pltpu.async_copy(
                table_hbm.at[lax.shift_right_logical(t, 3),
                             pl.ds(lax.bitwise_and(t, 7), 1)],
                rows_v.at[pl.ds(j + i, 1)], sem)

    def add_slice(rows_v, r0, nr):
        def row_body(r, c2):
            for cc in range(EMBED // 16):
                sl = pl.ds(cc * 16, 16)
                rows_v[r, sl] = rows_v[r, sl] + pos_v[r, sl]
            return c2

        lax.fori_loop(r0, r0 + nr, row_body, 0)

    def fire_interleaved(idx_f, rows_f, sem_f, rows_add):
        j = 0
        r0 = 0
        for gsz, nr in zip(GROUPS, ADD_SPLITS):
            fire_group(idx_f, rows_f, sem_f, j, gsz)
            add_slice(rows_add, r0, nr)
            j += gsz
            r0 += nr

    def fire_all(idx_v, rows_v, sem):
        j = 0
        for gsz in GROUPS:
            fire_group(idx_v, rows_v, sem, j, gsz)
            j += gsz

    def drain_rows(rows_v, sem):
        pltpu.make_async_copy(
            table_hbm.at[0, pl.ds(0, 1)], rows_v.at[pl.ds(0, CHUNK)], sem
        ).wait()

    def idx_fetch(c, idx_v, sem):
        c = jnp.minimum(c, last)
        pltpu.async_copy(tokens_hbm.at[pl.ds(base + c * CHUNK, CHUNK)],
                         idx_v.at[pl.ds(0, CHUNK)], sem)

    def idx_wait(idx_v, sem):
        pltpu.make_async_copy(tokens_hbm.at[pl.ds(0, CHUNK)],
                              idx_v.at[pl.ds(0, CHUNK)], sem).wait()

    def out_start(rows_v, c, sem):
        pltpu.async_copy(rows_v, out_hbm.at[pl.ds(base + c * CHUNK, CHUNK)], sem)

    def out_wait(rows_v, sem):
        pltpu.make_async_copy(rows_v, out_hbm.at[pl.ds(0, CHUNK)], sem).wait()

    pltpu.sync_copy(tokens_hbm.at[pl.ds(base, CHUNK)], idx_a.at[pl.ds(0, CHUNK)])
    fire_all(idx_a, rows_a, sem_ra)
    idx_fetch(1, idx_b, sem_ib)

    def pair_body(g, carry):
        ca = 2 * g

        idx_wait(idx_b, sem_ib)

        @pl.when(g > 0)
        def _():
            out_wait(rows_b, sem_ob)

        idx_fetch(ca + 2, idx_a, sem_ia)
        drain_rows(rows_a, sem_ra)
        fire_interleaved(idx_b, rows_b, sem_rb, rows_a)
        out_start(rows_a, ca, sem_oa)

        idx_wait(idx_a, sem_ia)
        out_wait(rows_a, sem_oa)
        idx_fetch(ca + 3, idx_b, sem_ib)
        drain_rows(rows_b, sem_rb)
        fire_interleaved(idx_a, rows_a, sem_ra, rows_b)
        out_start(rows_b, ca + 1, sem_ob)
        return carry

    lax.fori_loop(0, NCHUNKS // 2, pair_body, 0)

    idx_wait(idx_b, sem_ib)
    drain_rows(rows_a, sem_ra)
    out_wait(rows_b, sem_ob)


def kernel(tokens, input_embedding, position_embedding):
    flat = tokens.reshape(-1).astype(jnp.int32)
    table3 = input_embedding.reshape(VOCAB // 8, 8, EMBED)
    out = _embed_sc(flat, table3, position_embedding)
    return out.reshape(BATCH, NTOKENS, EMBED)


# speedup vs baseline: 3.2422x; 1.0118x over previous
import functools

import jax
import jax.numpy as jnp
from jax import lax
from jax.experimental import pallas as pl
from jax.experimental.pallas import tpu as pltpu
from jax.experimental.pallas import tpu_sc as plsc

VOCAB = 1000000
EMBED = 64
NTOKENS = 200
BATCH = 4096

TOTAL = BATCH * NTOKENS
NUM_WORKERS = 32
PER_WORKER = TOTAL // NUM_WORKERS
CHUNK = NTOKENS
NCHUNKS = PER_WORKER // CHUNK
GROUPS = [16] * 12 + [8]
ADD_SPLITS = [15] * 12 + [20]

_mesh = plsc.VectorSubcoreMesh(core_axis_name="c", subcore_axis_name="s")


@functools.partial(
    pl.kernel,
    mesh=_mesh,
    out_type=jax.ShapeDtypeStruct((TOTAL, EMBED), jnp.float32),
    scratch_types=[
        pltpu.VMEM((CHUNK + 8,), jnp.int32),
        pltpu.VMEM((CHUNK + 8,), jnp.int32),
        pltpu.VMEM((CHUNK, EMBED), jnp.float32),
        pltpu.VMEM((CHUNK, EMBED), jnp.float32),
        pltpu.VMEM((CHUNK, EMBED), jnp.float32),
        pltpu.SemaphoreType.DMA,
        pltpu.SemaphoreType.DMA,
        pltpu.SemaphoreType.DMA,
        pltpu.SemaphoreType.DMA,
        pltpu.SemaphoreType.DMA,
        pltpu.SemaphoreType.DMA,
    ],
    compiler_params=pltpu.CompilerParams(use_tc_tiling_on_sc=True),
)
def _embed_sc(tokens_hbm, table_hbm, pos_hbm, out_hbm,
              idx_a, idx_b, rows_a, rows_b, pos_v,
              sem_ia, sem_ib, sem_ra, sem_rb, sem_oa, sem_ob):
    wid = lax.axis_index("s") * 2 + lax.axis_index("c")
    base = wid * PER_WORKER
    last = NCHUNKS - 1

    pltpu.sync_copy(pos_hbm, pos_v)

    def extract_group(idx_v, j, gsz):
        v = idx_v[pl.ds(j, 16)]
        return [v[i] for i in range(gsz)]

    def enqueue_group(rows_v, sem, j, ts):
        for i, t in enumerate(ts):
            pltpu.async_copy(
                table_hbm.at[lax.shift_right_logical(t, 3),
                             pl.ds(lax.bitwise_and(t, 7), 1)],
                rows_v.at[pl.ds(j + i, 1)], sem)

    def add_slice(rows_v, r0, nr):
        def row_body(r, c2):
            for cc in range(EMBED // 16):
                sl = pl.ds(cc * 16, 16)
                rows_v[r, sl] = rows_v[r, sl] + pos_v[r, sl]
            return c2

        lax.fori_loop(r0, r0 + nr, row_body, 0)

    def fire_interleaved(idx_f, rows_f, sem_f, rows_add):
        offs = [0]
        for gsz in GROUPS:
            offs.append(offs[-1] + gsz)
        ts = extract_group(idx_f, 0, GROUPS[0])
        r0 = 0
        for k, (gsz, nr) in enumerate(zip(GROUPS, ADD_SPLITS)):
            ts_next = (extract_group(idx_f, offs[k + 1], GROUPS[k + 1])
                       if k + 1 < len(GROUPS) else None)
            enqueue_group(rows_f, sem_f, offs[k], ts)
            add_slice(rows_add, r0, nr)
            ts = ts_next
            r0 += nr

    def fire_all(idx_v, rows_v, sem):
        j = 0
        for gsz in GROUPS:
            enqueue_group(rows_v, sem, j, extract_group(idx_v, j, gsz))
            j += gsz

    def drain_rows(rows_v, sem):
        pltpu.make_async_copy(
            table_hbm.at[0, pl.ds(0, 1)], rows_v.at[pl.ds(0, CHUNK)], sem
        ).wait()

    def idx_fetch(c, idx_v, sem):
        c = jnp.minimum(c, last)
        pltpu.async_copy(tokens_hbm.at[pl.ds(base + c * CHUNK, CHUNK)],
                         idx_v.at[pl.ds(0, CHUNK)], sem)

    def idx_wait(idx_v, sem):
        pltpu.make_async_copy(tokens_hbm.at[pl.ds(0, CHUNK)],
                              idx_v.at[pl.ds(0, CHUNK)], sem).wait()

    def out_start(rows_v, c, sem):
        pltpu.async_copy(rows_v, out_hbm.at[pl.ds(base + c * CHUNK, CHUNK)], sem)

    def out_wait(rows_v, sem):
        pltpu.make_async_copy(rows_v, out_hbm.at[pl.ds(0, CHUNK)], sem).wait()

    pltpu.sync_copy(tokens_hbm.at[pl.ds(base, CHUNK)], idx_a.at[pl.ds(0, CHUNK)])
    fire_all(idx_a, rows_a, sem_ra)
    idx_fetch(1, idx_b, sem_ib)

    def pair_body(g, carry):
        ca = 2 * g

        idx_wait(idx_b, sem_ib)

        @pl.when(g > 0)
        def _():
            out_wait(rows_b, sem_ob)

        idx_fetch(ca + 2, idx_a, sem_ia)
        drain_rows(rows_a, sem_ra)
        fire_interleaved(idx_b, rows_b, sem_rb, rows_a)
        out_start(rows_a, ca, sem_oa)

        idx_wait(idx_a, sem_ia)
        out_wait(rows_a, sem_oa)
        idx_fetch(ca + 3, idx_b, sem_ib)
        drain_rows(rows_b, sem_rb)
        fire_interleaved(idx_a, rows_a, sem_ra, rows_b)
        out_start(rows_b, ca + 1, sem_ob)
        return carry

    lax.fori_loop(0, NCHUNKS // 2, pair_body, 0)

    idx_wait(idx_b, sem_ib)
    drain_rows(rows_a, sem_ra)
    out_wait(rows_b, sem_ob)


def kernel(tokens, input_embedding, position_embedding):
    flat = tokens.reshape(-1).astype(jnp.int32)
    table3 = input_embedding.reshape(VOCAB // 8, 8, EMBED)
    out = _embed_sc(flat, table3, position_embedding)
    return out.reshape(BATCH, NTOKENS, EMBED)
